# baseline (speedup 1.0000x reference)
import functools

import jax
import jax.numpy as jnp
from jax import lax
from jax.experimental import pallas as pl
from jax.experimental.pallas import tpu as pltpu
from jax.experimental.pallas import tpu_sc as plsc

_BATCH = 16384
_V = 1000000
_K = 16
_NC = 2
_NS = 16
_KH = _K // _NC
_BPT = _BATCH // _NS
_NCH = _BPT // 128

_L = 110592
_TS = _L // _NS
_NSTEP = 9
_RLO = _NSTEP * _L
_RN = 4608
_TLO = _RLO + _RN

_mesh = plsc.VectorSubcoreMesh(core_axis_name="c", subcore_axis_name="s")


@functools.partial(
    pl.kernel,
    mesh=_mesh,
    out_type=jax.ShapeDtypeStruct((_NC, _NS, _K), jnp.float32),
    scratch_types=[
        [pltpu.VMEM_SHARED((_L,), jnp.float32) for _ in range(_KH)],
        [pltpu.VMEM_SHARED((_L,), jnp.float32) for _ in range(_KH)],
        [pltpu.VMEM_SHARED((128,), jnp.float32) for _ in range(_KH)],
        pltpu.VMEM_SHARED((_NS, _K), jnp.float32),
        pltpu.VMEM((_NCH, 128), jnp.int32),
        pltpu.VMEM((_KH, _BPT), jnp.float32),
        pltpu.VMEM((_NCH, 128), jnp.int32),
        pltpu.VMEM((_KH, _BPT), jnp.float32),
        pltpu.VMEM((_K,), jnp.float32),
        pltpu.SemaphoreType.DMA,
        pltpu.SemaphoreType.DMA,
        pltpu.SemaphoreType.DMA,
    ],
)
def _sc_loss(embt_hbm, idx_hbm, tablet_hbm, tail_hbm, out_hbm,
             spa, spb, spt, spp, idx_v, emb_v, adj_v, gt_v, acc_v,
             ssem, gsem, esem):
    c = lax.axis_index("c")
    s = lax.axis_index("s")
    klo = c * _KH
    base = s * _BPT

    pltpu.sync_copy(idx_hbm.at[s], idx_v)
    emb_cp = pltpu.async_copy(
        embt_hbm.at[pl.ds(klo, _KH), pl.ds(base, _BPT)], emb_v, esem
    )

    @pl.when(s == 0)
    def _():
        for k in range(_KH):
            pltpu.sync_copy(tail_hbm.at[klo + k], spt[k])

    def stage_final_window():
        @pl.when(s < _NS - 1)
        def _():
            for k in range(_KH):
                pltpu.sync_copy(
                    tablet_hbm.at[klo + k, pl.ds(_RLO + s * 256, 256)],
                    spb[k].at[pl.ds(s * 256, 256)],
                )

        @pl.when(s == _NS - 1)
        def _():
            for k in range(_KH):
                pltpu.sync_copy(
                    tablet_hbm.at[klo + k, pl.ds(_RLO + 3840, 768)],
                    spb[k].at[pl.ds(3840, 768)],
                )

    def issue_stream(q):
        dst = spa if q % 2 == 0 else spb
        lo = q * _L
        return [
            pltpu.async_copy(
                tablet_hbm.at[klo + k, pl.ds(lo + s * _TS, _TS)],
                dst[k].at[pl.ds(s * _TS, _TS)], ssem,
            )
            for k in range(_KH)
        ]

    def compute_window(sp, lo, lq, acc):
        def adj_body(m, carry):
            ch = m >> 3
            j16 = (m & 7) * 16
            v = idx_v[ch, pl.ds(j16, 16)]
            adj_v[ch, pl.ds(j16, 16)] = jnp.minimum(
                jnp.maximum(v - lo, 0), lq - 1
            )
            return carry

        lax.fori_loop(0, _NCH * 8, adj_body, 0)
        cps = [
            pltpu.async_copy(
                sp[k].at[adj_v.at[ch]],
                gt_v.at[k, pl.ds(ch * 128, 128)],
                gsem,
            )
            for ch in range(_NCH)
            for k in range(_KH)
        ]
        for cp in cps:
            cp.wait()

        def acc_body(m, a):
            ch = m >> 3
            j16 = (m & 7) * 16
            v = idx_v[ch, pl.ds(j16, 16)]
            valid = (v >= lo) & (v < lo + lq)
            for k in range(_KH):
                d = (
                    emb_v[k, pl.ds(m * 16, 16)]
                    - gt_v[k, pl.ds(m * 16, 16)]
                )
                a = a + jnp.where(valid, d * d, 0.0)
            return a

        return lax.fori_loop(0, _NCH * 8, acc_body, acc)

    cps_q = issue_stream(0)
    emb_cp.wait()

    acc = jnp.zeros((16,), jnp.float32)
    for q in range(_NSTEP):
        for cp in cps_q:
            cp.wait()
        plsc.subcore_barrier()
        if q + 1 < _NSTEP:
            cps_q = issue_stream(q + 1)
        if q == _NSTEP - 1:
            stage_final_window()
        sp = spa if q % 2 == 0 else spb
        acc = compute_window(sp, q * _L, _L, acc)
        plsc.subcore_barrier()

    acc = compute_window(spb, _RLO, _RN, acc)
    acc = compute_window(spt, _TLO, _V - _TLO, acc)

    acc_v[...] = acc
    pltpu.sync_copy(acc_v, spp.at[s])
    plsc.subcore_barrier()

    @pl.when(s == 0)
    def _():
        pltpu.sync_copy(spp, out_hbm.at[c])


def kernel(embeddings, labels, table):
    idx = (labels.astype(jnp.int32) - 1).reshape(_NS, _NCH, 128)
    tablet = table.T
    tail = jnp.pad(tablet[:, _TLO:], ((0, 0), (0, 128 - (_V - _TLO))))
    partials = _sc_loss(embeddings.T, idx, tablet, tail)
    return jnp.sum(partials)


# speedup vs baseline: 1.4108x; 1.4108x over previous
import functools

import jax
import jax.numpy as jnp
from jax import lax
from jax.experimental import pallas as pl
from jax.experimental.pallas import tpu as pltpu
from jax.experimental.pallas import tpu_sc as plsc

_BATCH = 16384
_K = 16
_NC = 2
_NS = 16
_NW = _NC * _NS
_BPW = _BATCH // _NW
_CHUNK = 128
_NCHUNK = _BPW // _CHUNK

_mesh = plsc.VectorSubcoreMesh(core_axis_name="c", subcore_axis_name="s")


@functools.partial(
    pl.kernel,
    mesh=_mesh,
    compiler_params=pltpu.CompilerParams(use_tc_tiling_on_sc=False),
    out_type=jax.ShapeDtypeStruct((_NW, _K), jnp.float32),
    scratch_types=[
        pltpu.VMEM((_NCHUNK, _CHUNK), jnp.int32),
        pltpu.VMEM((_BPW, _K), jnp.float32),
        pltpu.VMEM((_BPW, _K), jnp.float32),
        pltpu.VMEM((_K,), jnp.float32),
        pltpu.SemaphoreType.DMA,
        pltpu.SemaphoreType.DMA,
    ],
)
def _sc_loss(emb_hbm, idx_hbm, table_hbm, out_hbm,
             idx_v, rows_v, emb_v, acc_v, gsem, esem):
    wid = lax.axis_index("s") * _NC + lax.axis_index("c")
    base = wid * _BPW

    pltpu.sync_copy(idx_hbm.at[wid], idx_v)
    emb_cp = pltpu.async_copy(emb_hbm.at[pl.ds(base, _BPW)], emb_v, esem)
    gathers = []
    for j in range(_NCHUNK):
        gathers.append(
            pltpu.async_copy(
                table_hbm.at[idx_v.at[j]],
                rows_v.at[pl.ds(j * _CHUNK, _CHUNK)],
                gsem,
            )
        )
    emb_cp.wait()
    for cp in gathers:
        cp.wait()

    def body(i, acc):
        d = emb_v[i, :] - rows_v[i, :]
        return acc + d * d

    acc = lax.fori_loop(0, _BPW, body, jnp.zeros((_K,), jnp.float32))
    acc_v[...] = acc
    pltpu.sync_copy(acc_v, out_hbm.at[wid])


def kernel(embeddings, labels, table):
    idx = (labels.astype(jnp.int32) - 1).reshape(_NW, _NCHUNK, _CHUNK)
    partials = _sc_loss(embeddings, idx, table)
    return jnp.sum(partials)
